# Initial kernel scaffold; baseline (speedup 1.0000x reference)
import jax
import jax.numpy as jnp
from jax.experimental import pallas as pl


def kernel(x, edge_index, edge_type, batch, enc_W, enc_b, prelu_a, conv_Wrel, conv_Wroot, conv_b, gp_W1, gp_b1, gp_W2, gp_b2, fc_W1, fc_b1, fc_W2, fc_b2, out_W, out_b):
    raise NotImplementedError("write your pallas kernel here")


# speedup vs baseline: 13.4329x; 13.4329x over previous
import functools

import jax
import jax.numpy as jnp
from jax import lax
from jax.experimental import pallas as pl
from jax.experimental.pallas import tpu as pltpu
from jax.experimental.pallas import tpu_sc as plsc


def _mm_bias_body(h_ref, w_ref, b_ref, o_ref):
  o_ref[...] = (
      jnp.dot(h_ref[...], w_ref[...], preferred_element_type=jnp.float32)
      + b_ref[...]
  )


def _mm_bias(h, w, b2d, bn):
  n, k = h.shape
  m = w.shape[1]
  return pl.pallas_call(
      _mm_bias_body,
      grid=(n // bn,),
      in_specs=[
          pl.BlockSpec((bn, k), lambda j: (j, 0)),
          pl.BlockSpec((k, m), lambda j: (0, 0)),
          pl.BlockSpec((1, m), lambda j: (0, 0)),
      ],
      out_specs=pl.BlockSpec((bn, m), lambda j: (j, 0)),
      out_shape=jax.ShapeDtypeStruct((n, m), jnp.float32),
  )(h, w, b2d)


def _mm_cat(h, w, b2d, bn):
  n, d = h.shape
  m = w.shape[1]
  nkc = m // 128
  nj = n // bn
  return pl.pallas_call(
      _mm_bias_body,
      grid=(nj, nkc),
      in_specs=[
          pl.BlockSpec((bn, d), lambda j, k: (j, 0)),
          pl.BlockSpec((d, 128), lambda j, k: (0, k)),
          pl.BlockSpec((1, 128), lambda j, k: (0, k)),
      ],
      out_specs=pl.BlockSpec((bn, 128), lambda j, k: (k * nj + j, 0)),
      out_shape=jax.ShapeDtypeStruct((n * nkc, 128), jnp.float32),
  )(h, w, b2d)


def _combine_body(r0_ref, r1_ref, agg_ref, h_ref, a_ref, o_ref):
  root = jnp.concatenate([r0_ref[...], r1_ref[...]], axis=1)
  v = root + agg_ref[...] + h_ref[...]
  a = a_ref[0, 0]
  v = jnp.maximum(v, 0.0) + a * jnp.minimum(v, 0.0)
  n2 = jnp.sum(v * v, axis=1, keepdims=True)
  o_ref[...] = v * lax.rsqrt(jnp.maximum(n2, 1e-24))


def _combine(x2, agg, h, a2d, bn, d):
  n = h.shape[0]
  nj = n // bn
  return pl.pallas_call(
      _combine_body,
      grid=(nj,),
      in_specs=[
          pl.BlockSpec((bn, 128), lambda j: (j, 0)),
          pl.BlockSpec((bn, 128), lambda j: (nj + j, 0)),
          pl.BlockSpec((bn, d), lambda j: (j, 0)),
          pl.BlockSpec((bn, d), lambda j: (j, 0)),
          pl.BlockSpec((1, 1), lambda j: (0, 0), memory_space=pltpu.SMEM),
      ],
      out_specs=pl.BlockSpec((bn, d), lambda j: (j, 0)),
      out_shape=jax.ShapeDtypeStruct((n, d), jnp.float32),
  )(x2, x2, agg, h, a2d)


def _gp_pool_body(h_ref, w1_ref, b1_ref, w2_ref, b2_ref, batch_ref, o_ref):
  j = pl.program_id(0)
  t = jnp.maximum(
      jnp.dot(h_ref[...], w1_ref[...], preferred_element_type=jnp.float32)
      + b1_ref[...], 0.0)
  t = jnp.dot(t, w2_ref[...], preferred_element_type=jnp.float32) + b2_ref[...]
  bn, g = h_ref.shape[0], o_ref.shape[0]
  onehot = (batch_ref[...] ==
            lax.broadcasted_iota(jnp.int32, (bn, g), 1)).astype(jnp.float32)
  part = lax.dot_general(onehot, t, (((0,), (0,)), ((), ())),
                         preferred_element_type=jnp.float32)

  @pl.when(j == 0)
  def _():
    o_ref[...] = part

  @pl.when(j > 0)
  def _():
    o_ref[...] = o_ref[...] + part


def _gp_pool(h, w1, b1_2d, w2, b2_2d, batch2d, g, bn):
  n, d = h.shape
  return pl.pallas_call(
      _gp_pool_body,
      grid=(n // bn,),
      in_specs=[
          pl.BlockSpec((bn, d), lambda j: (j, 0)),
          pl.BlockSpec((d, d), lambda j: (0, 0)),
          pl.BlockSpec((1, d), lambda j: (0, 0)),
          pl.BlockSpec((d, d), lambda j: (0, 0)),
          pl.BlockSpec((1, d), lambda j: (0, 0)),
          pl.BlockSpec((bn, 1), lambda j: (j, 0)),
      ],
      out_specs=pl.BlockSpec((g, d), lambda j: (0, 0)),
      out_shape=jax.ShapeDtypeStruct((g, d), jnp.float32),
  )(h, w1, b1_2d, w2, b2_2d, batch2d)


def _head_body(g_ref, w1_ref, b1_ref, w2_ref, b2_ref, w3_ref, b3_ref, o_ref):
  a = jnp.maximum(
      jnp.dot(g_ref[...], w1_ref[...], preferred_element_type=jnp.float32)
      + b1_ref[...], 0.0)
  b = jnp.maximum(
      jnp.dot(a, w2_ref[...], preferred_element_type=jnp.float32)
      + b2_ref[...], 0.0)
  o_ref[...] = jnp.maximum(
      jnp.dot(b, w3_ref[...], preferred_element_type=jnp.float32)
      + b3_ref[...], 0.0)


def _head(g, w1, b1_2d, w2, b2_2d, w3p, b3p):
  gg = g.shape[0]
  return pl.pallas_call(
      _head_body,
      out_shape=jax.ShapeDtypeStruct((gg, 128), jnp.float32),
  )(g, w1, b1_2d, w2, b2_2d, w3p, b3p)


_SC_NSUB = 16
_SC_CHUNK = 128


def _make_sc_agg(n, n_chunks, acc_rows):
  zrows = acc_rows // _SC_NSUB
  tail = n - (_SC_NSUB - 1) * zrows
  assert zrows % 128 == 0 and tail % 8 == 0 and 0 < tail <= zrows
  mesh = plsc.VectorSubcoreMesh(core_axis_name="c", subcore_axis_name="s")

  @functools.partial(
      pl.kernel,
      out_type=jax.ShapeDtypeStruct((n, 256), jnp.float32),
      mesh=mesh,
      scratch_types=dict(
          acc=pltpu.VMEM_SHARED((acc_rows, 128), jnp.float32),
          gidx_v=pltpu.VMEM((n_chunks, _SC_CHUNK), jnp.int32),
          dst_v=pltpu.VMEM((n_chunks, _SC_CHUNK), jnp.int32),
          rows_v=pltpu.VMEM((_SC_CHUNK, 128), jnp.float32),
          sem=pltpu.SemaphoreType.DMA,
      ),
  )
  def agg_kernel(x2_hbm, gidx_hbm, dst_hbm, out_hbm, acc, gidx_v, dst_v,
                 rows_v, sem):
    c = lax.axis_index("c")
    s = lax.axis_index("s")

    zv = jnp.zeros((16,), jnp.float32)

    def zrow(i, _):
      for jj in range(8):
        rows_v[i, pl.ds(jj * 16, 16)] = zv
      return 0

    lax.fori_loop(0, _SC_CHUNK, zrow, 0)

    for b in range(zrows // _SC_CHUNK):
      pltpu.sync_copy(rows_v, acc.at[pl.ds(s * zrows + b * _SC_CHUNK, _SC_CHUNK)])

    pltpu.sync_copy(gidx_hbm.at[c, s], gidx_v)
    pltpu.sync_copy(dst_hbm.at[s], dst_v)

    plsc.subcore_barrier()

    def chunk(j, _):
      pltpu.async_copy(x2_hbm.at[gidx_v.at[j]], rows_v, sem).wait()
      pltpu.sync_copy(rows_v, acc.at[dst_v.at[j]], add=True)
      return 0

    lax.fori_loop(0, n_chunks, chunk, 0)

    plsc.subcore_barrier()

    @pl.when(s < _SC_NSUB - 1)
    def _():
      pltpu.sync_copy(
          acc.at[pl.ds(s * zrows, zrows)],
          out_hbm.at[pl.ds(s * zrows, zrows), pl.ds(c * 128, 128)])

    @pl.when(s == _SC_NSUB - 1)
    def _():
      pltpu.sync_copy(
          acc.at[pl.ds((_SC_NSUB - 1) * zrows, tail)],
          out_hbm.at[pl.ds((_SC_NSUB - 1) * zrows, tail), pl.ds(c * 128, 128)])

  return agg_kernel


def kernel(x, edge_index, edge_type, batch, enc_W, enc_b, prelu_a,
           conv_Wrel, conv_Wroot, conv_b,
           gp_W1, gp_b1, gp_W2, gp_b2,
           fc_W1, fc_b1, fc_W2, fc_b2, out_W, out_b):
  n, f_in = x.shape
  e = edge_type.shape[0]
  l_layers, r_rel, d, _ = conv_Wrel.shape
  g = 32
  bn = 2000

  src = edge_index[0]
  dst = edge_index[1]

  n_half_cols = 2 * (1 + r_rel)
  per_sub = e // _SC_NSUB
  n_chunks = -(-per_sub // _SC_CHUNK)
  pad = n_chunks * _SC_CHUNK - per_sub
  acc_rows = ((n + _SC_NSUB * 128 - 1) // (_SC_NSUB * 128)) * (_SC_NSUB * 128)

  gbase = (2 + 2 * edge_type) * n + src
  gbase = gbase.reshape(_SC_NSUB, per_sub)
  gbase = jnp.pad(gbase, ((0, 0), (0, pad)))
  gidx = jnp.stack([gbase, gbase + n])
  gidx = gidx.reshape(2, _SC_NSUB, n_chunks, _SC_CHUNK)

  dstp = dst.reshape(_SC_NSUB, per_sub)
  dstp = jnp.pad(dstp, ((0, 0), (0, pad)), constant_values=n)
  dstp = dstp.reshape(_SC_NSUB, n_chunks, _SC_CHUNK)

  wcat = jnp.concatenate(
      [conv_Wroot, conv_Wrel.transpose(0, 2, 1, 3).reshape(l_layers, d, r_rel * d)],
      axis=2)
  bcat = jnp.concatenate(
      [conv_b, jnp.zeros((l_layers, r_rel * d), jnp.float32)], axis=1)

  a2d = jnp.full((1, 1), prelu_a, jnp.float32)
  batch2d = batch.reshape(n, 1)

  sc_agg = _make_sc_agg(n, n_chunks, acc_rows)

  h = _mm_bias(x, enc_W, enc_b.reshape(1, -1), bn)

  for i in range(l_layers):
    x2 = _mm_cat(h, wcat[i], bcat[i].reshape(1, -1), bn)
    agg = sc_agg(x2, gidx, dstp)
    h = _combine(x2, agg, h, a2d, 2000, d)

  gpool = _gp_pool(h, gp_W1, gp_b1.reshape(1, -1), gp_W2, gp_b2.reshape(1, -1),
                   batch2d, g, 2000)
  w3p = jnp.pad(out_W, ((0, 0), (0, 127)))
  b3p = jnp.pad(out_b, (0, 127)).reshape(1, 128)
  out = _head(gpool, fc_W1, fc_b1.reshape(1, -1), fc_W2, fc_b2.reshape(1, -1),
              w3p, b3p)
  return out[:, :1]
